# baseline (speedup 1.0000x reference)
import functools
import math

import jax
import jax.numpy as jnp
from jax import lax
from jax.experimental import pallas as pl
from jax.experimental.pallas import tpu as pltpu
from jax.experimental.pallas import tpu_sc as plsc

N = 10000
D = 128
NC = 2
NS = 16
NW = NC * NS
L = 16
CH = 128

_mesh = functools.partial(
    plsc.VectorSubcoreMesh,
    core_axis_name="c", subcore_axis_name="s",
    num_cores=NC, num_subcores=NS,
)


def _wid():
  return lax.axis_index("s") * NC + lax.axis_index("c")


def _pre_body(emb, Wg, asr, adr, Wq, bq, Wk, bk, Wv, bv, Ws, bs,
              h_o, q_o, k_o, v_o, skip_o, a_src_o, a_dst_o, selfw_o):
  e = emb[...]
  h = jnp.dot(e, Wg[...], preferred_element_type=jnp.float32)
  h_o[...] = h
  a_s = jnp.sum(h * asr[...], axis=1, keepdims=True)
  a_d = jnp.sum(h * adr[...], axis=1, keepdims=True)
  a_src_o[...] = a_s
  a_dst_o[...] = a_d
  ls = a_s + a_d
  selfw_o[...] = jnp.exp(jnp.where(ls >= 0, ls, 0.2 * ls))
  q_o[...] = jnp.dot(e, Wq[...], preferred_element_type=jnp.float32) + bq[...]
  k_o[...] = jnp.dot(e, Wk[...], preferred_element_type=jnp.float32) + bk[...]
  v_o[...] = jnp.dot(e, Wv[...], preferred_element_type=jnp.float32) + bv[...]
  skip_o[...] = jnp.dot(e, Ws[...], preferred_element_type=jnp.float32) + bs[...]


def _dense_pre(emb, Wg, att_src, att_dst, Wq, bq, Wk, bk, Wv, bv, Ws, bs):
  R = 1000
  grid = (N // R,)
  row_blk = pl.BlockSpec((R, D), lambda i: (i, 0))
  full = pl.BlockSpec((D, D), lambda i: (0, 0))
  vec = pl.BlockSpec((1, D), lambda i: (0, 0))
  col = pl.BlockSpec((R, 1), lambda i: (i, 0))
  f32 = jnp.float32
  return pl.pallas_call(
      _pre_body,
      grid=grid,
      in_specs=[row_blk, full, vec, vec, full, vec, full, vec, full, vec,
                full, vec],
      out_specs=[row_blk, row_blk, row_blk, row_blk, row_blk, col, col, col],
      out_shape=[
          jax.ShapeDtypeStruct((N, D), f32),
          jax.ShapeDtypeStruct((N, D), f32),
          jax.ShapeDtypeStruct((N, D), f32),
          jax.ShapeDtypeStruct((N, D), f32),
          jax.ShapeDtypeStruct((N, D), f32),
          jax.ShapeDtypeStruct((N, 1), f32),
          jax.ShapeDtypeStruct((N, 1), f32),
          jax.ShapeDtypeStruct((N, 1), f32),
      ],
  )(emb, Wg, att_src.reshape(1, D), att_dst.reshape(1, D), Wq,
    bq.reshape(1, D), Wk, bk.reshape(1, D), Wv, bv.reshape(1, D), Ws,
    bs.reshape(1, D))


def _gat_edge_sc(a_src, a_dst, src, dst):
  E = src.shape[0]
  ept = E // NW
  f32 = jnp.float32

  @functools.partial(
      pl.kernel,
      out_type=(jax.ShapeDtypeStruct((E,), f32),
                jax.ShapeDtypeStruct((NW, 1, N), f32)),
      mesh=_mesh(),
      compiler_params=pltpu.CompilerParams(needs_layout_passes=False),
      scratch_types=[
          pltpu.VMEM((ept,), jnp.int32),
          pltpu.VMEM((ept,), jnp.int32),
          pltpu.VMEM((ept,), f32),
          pltpu.VMEM((N,), f32),
          pltpu.VMEM((N,), f32),
          pltpu.VMEM((N,), f32),
      ],
  )
  def k(a_src_h, a_dst_h, src_h, dst_h, w_h, denp_h,
        src_v, dst_v, w_v, as_v, ad_v, den_v):
    wid = _wid()
    base = wid * ept
    pltpu.sync_copy(src_h.at[pl.ds(base, ept)], src_v)
    pltpu.sync_copy(dst_h.at[pl.ds(base, ept)], dst_v)
    pltpu.sync_copy(a_src_h, as_v)
    pltpu.sync_copy(a_dst_h, ad_v)

    zero16 = jnp.zeros((L,), f32)

    def zero_body(i, _):
      den_v[pl.ds(i * L, L)] = zero16
      return 0
    lax.fori_loop(0, N // L, zero_body, 0)

    def body(g, _):
      s16 = src_v[pl.ds(g * L, L)]
      d16 = dst_v[pl.ds(g * L, L)]
      av = plsc.load_gather(as_v, [s16])
      bv = plsc.load_gather(ad_v, [d16])
      lg = av + bv
      lg = jnp.where(lg >= 0, lg, 0.2 * lg)
      wv = jnp.exp(lg)
      w_v[pl.ds(g * L, L)] = wv
      plsc.addupdate_scatter(den_v, [d16], wv)
      return 0
    lax.fori_loop(0, ept // L, body, 0)

    pltpu.sync_copy(w_v, w_h.at[pl.ds(base, ept)])
    pltpu.sync_copy(den_v, denp_h.at[wid, 0])

  return k(a_src, a_dst, src, dst)


def _trans_edge_sc(qp, kp, src, dst):
  E = src.shape[0]
  nch = E // CH
  base_cnt = nch // NW
  extra = nch - base_cnt * NW
  inv_sqrt_d = 1.0 / math.sqrt(D)
  f32 = jnp.float32

  NB = 3
  niter = (base_cnt + 1 + NB - 1) // NB

  @functools.partial(
      pl.kernel,
      out_type=(jax.ShapeDtypeStruct((E,), f32),
                jax.ShapeDtypeStruct((NW, 1, N), f32)),
      mesh=_mesh(),
      compiler_params=pltpu.CompilerParams(needs_layout_passes=False),
      scratch_types=[
          pltpu.VMEM((NB, CH), jnp.int32),
          pltpu.VMEM((NB, CH), jnp.int32),
          pltpu.VMEM((NB, CH, D), f32),
          pltpu.VMEM((NB, CH, D), f32),
          pltpu.VMEM((CH,), f32),
          pltpu.VMEM((N,), f32),
          pltpu.SemaphoreType.DMA,
          pltpu.SemaphoreType.DMA,
          pltpu.SemaphoreType.DMA,
      ],
  )
  def k(qp_h, kp_h, src_h, dst_h, tw_h, denp_h,
        si_v, di_v, qr_v, kr_v, tw_v, den_v, sem0, sem1, sem2):
    sems = (sem0, sem1, sem2)
    wid = _wid()
    zero16 = jnp.zeros((L,), f32)

    def zero_body(i, _):
      den_v[pl.ds(i * L, L)] = zero16
      return 0
    lax.fori_loop(0, N // L, zero_body, 0)

    my_cnt = base_cnt + jnp.where(wid < extra, 1, 0)
    iota = lax.iota(jnp.int32, L)

    def issue(c, b):
      @pl.when(c < my_cnt)
      def _():
        cb = (wid + c * NW) * CH
        pltpu.sync_copy(src_h.at[pl.ds(cb, CH)], si_v.at[b])
        pltpu.sync_copy(dst_h.at[pl.ds(cb, CH)], di_v.at[b])
        pltpu.async_copy(qp_h.at[di_v.at[b]], qr_v.at[b], sems[b])
        pltpu.async_copy(kp_h.at[si_v.at[b]], kr_v.at[b], sems[b])

    def process(c, b):
      @pl.when(c < my_cnt)
      def _():
        cb = (wid + c * NW) * CH
        pltpu.make_async_copy(qp_h.at[di_v.at[b]], qr_v.at[b], sems[b]).wait()
        pltpu.make_async_copy(kp_h.at[si_v.at[b]], kr_v.at[b], sems[b]).wait()

        def group_body(g, _):
          tlv = jnp.zeros((L,), f32)
          for e in range(L):
            row = g * L + e
            acc = jnp.zeros((L,), f32)
            for j in range(D // L):
              sl = pl.ds(j * L, L)
              acc = acc + qr_v[b, row, sl] * kr_v[b, row, sl]
            tlv = jnp.where(iota == e, jnp.sum(acc), tlv)
          wv = jnp.exp(tlv * inv_sqrt_d)
          tw_v[pl.ds(g * L, L)] = wv
          d16 = di_v[b, pl.ds(g * L, L)]
          plsc.addupdate_scatter(den_v, [d16], wv)
          return 0
        lax.fori_loop(0, CH // L, group_body, 0)
        pltpu.sync_copy(tw_v, tw_h.at[pl.ds(cb, CH)])

    for b in range(NB - 1):
      issue(b, b)

    def chunk_body(t, _):
      for b in range(NB):
        c = t * NB + b
        issue(c + NB - 1, (b + NB - 1) % NB)
        process(c, b)
      return 0
    lax.fori_loop(0, niter, chunk_body, 0)

    pltpu.sync_copy(den_v, denp_h.at[wid, 0])

  return k(qp, kp, src, dst)


def _combine_body(dg, selfw, dt, inv_g_o, inv_t_o):
  sg = jnp.sum(dg[...], axis=0, keepdims=True) + selfw[...]
  st = jnp.sum(dt[...], axis=0, keepdims=True)
  inv_g_o[...] = 1.0 / (sg + 1e-16)
  inv_t_o[...] = 1.0 / (st + 1e-16)


def _combine(denp_g, selfw_row, denp_t):
  f32 = jnp.float32
  return pl.pallas_call(
      _combine_body,
      out_shape=[jax.ShapeDtypeStruct((1, N), f32),
                 jax.ShapeDtypeStruct((1, N), f32)],
  )(denp_g, selfw_row, denp_t)


def _agg_sc(rows, w, src, dst, zeros_nd):
  E = src.shape[0]
  CHA = 64
  nch = E // CHA
  base_cnt = nch // NW
  extra = nch - base_cnt * NW
  NB = 5
  niter = (base_cnt + 1 + NB - 1) // NB
  RB = 80
  nrb = N // RB
  rb_base = nrb // NS
  rb_extra = nrb - rb_base * NS
  f32 = jnp.float32

  @functools.partial(
      pl.kernel,
      out_type=jax.ShapeDtypeStruct((NC, N, D), f32),
      mesh=_mesh(),
      compiler_params=pltpu.CompilerParams(needs_layout_passes=False),
      scratch_types=[
          pltpu.VMEM((NB, CHA), jnp.int32),
          pltpu.VMEM((NB, CHA), jnp.int32),
          pltpu.VMEM((NB, CHA, D), f32),
          pltpu.VMEM((NB, CHA), f32),
          pltpu.VMEM_SHARED((N, D), f32),
          pltpu.SemaphoreType.DMA,
          pltpu.SemaphoreType.DMA,
          pltpu.SemaphoreType.DMA,
          pltpu.SemaphoreType.DMA,
          pltpu.SemaphoreType.DMA,
      ],
  )
  def k(rows_h, w_h, src_h, dst_h, zeros_h, out_h,
        si_v, di_v, r_v, w_v, acc, sem0, sem1, sem2, sem3, sem4):
    sems = (sem0, sem1, sem2, sem3, sem4)
    cid = lax.axis_index("c")
    sid = lax.axis_index("s")
    wid = sid * NC + cid

    def init_body(ci, _):
      rb = (sid + ci * NS) * RB
      pltpu.sync_copy(zeros_h.at[pl.ds(rb, RB)], acc.at[pl.ds(rb, RB)])
      return 0
    lax.fori_loop(0, rb_base + jnp.where(sid < rb_extra, 1, 0), init_body, 0)
    plsc.subcore_barrier()

    my_cnt = base_cnt + jnp.where(wid < extra, 1, 0)

    def issue(c, b):
      @pl.when(c < my_cnt)
      def _():
        cb = (wid + c * NW) * CHA
        pltpu.sync_copy(src_h.at[pl.ds(cb, CHA)], si_v.at[b])
        pltpu.sync_copy(dst_h.at[pl.ds(cb, CHA)], di_v.at[b])
        pltpu.sync_copy(w_h.at[pl.ds(cb, CHA)], w_v.at[b])
        pltpu.async_copy(rows_h.at[si_v.at[b]], r_v.at[b], sems[b])

    def process(c, b):
      @pl.when(c < my_cnt)
      def _():
        pltpu.make_async_copy(rows_h.at[si_v.at[b]], r_v.at[b],
                              sems[b]).wait()

        def scale_body(g, _):
          w16 = w_v[b, pl.ds(g * L, L)]
          for e in range(L):
            a = w16[e]
            row = g * L + e
            for j in range(D // L):
              sl = pl.ds(j * L, L)
              r_v[b, row, sl] = r_v[b, row, sl] * a
          return 0
        lax.fori_loop(0, CHA // L, scale_body, 0)

        pltpu.sync_copy(r_v.at[b], acc.at[di_v.at[b]], add=True)

    for b in range(NB - 1):
      issue(b, b)

    def chunk_body(t, _):
      for b in range(NB):
        c = t * NB + b
        issue(c + NB - 1, (b + NB - 1) % NB)
        process(c, b)
      return 0
    lax.fori_loop(0, niter, chunk_body, 0)

    plsc.subcore_barrier()

    def flush_body(ci, _):
      rb = (sid + ci * NS) * RB
      pltpu.sync_copy(acc.at[pl.ds(rb, RB)], out_h.at[cid].at[pl.ds(rb, RB)])
      return 0
    lax.fori_loop(0, rb_base + jnp.where(sid < rb_extra, 1, 0), flush_body, 0)

  return k(rows, w, src, dst, zeros_nd)


def _final_body(gp, tp, h, selfw, inv_g, inv_t, skip, bg, out_o):
  gat = (gp[0] + gp[1] + h[...] * selfw[...]) * inv_g[...] + bg[...]
  x1 = jnp.maximum(gat, 0.0)
  tout = (tp[0] + tp[1]) * inv_t[...] + skip[...]
  out = x1 + tout
  nrm = jnp.sqrt(jnp.sum(out * out, axis=1, keepdims=True))
  out_o[...] = out / jnp.maximum(nrm, 1e-12)


def _final(gat_p, t_p, h, selfw, inv_g, inv_t, skip, b_gat):
  R = 1000
  f32 = jnp.float32
  part_blk = pl.BlockSpec((NC, R, D), lambda i: (0, i, 0))
  row_blk = pl.BlockSpec((R, D), lambda i: (i, 0))
  col_blk = pl.BlockSpec((R, 1), lambda i: (i, 0))
  vec = pl.BlockSpec((1, D), lambda i: (0, 0))
  return pl.pallas_call(
      _final_body,
      grid=(N // R,),
      in_specs=[part_blk, part_blk, row_blk, col_blk, col_blk, col_blk,
                row_blk, vec],
      out_specs=row_blk,
      out_shape=jax.ShapeDtypeStruct((N, D), f32),
  )(gat_p, t_p, h, selfw, inv_g, inv_t, skip, b_gat.reshape(1, D))


@jax.jit
def kernel(x, edge, embedding, W_gat, att_src, att_dst, b_gat,
           Wq, bq, Wk, bk, Wv, bv, Wskip, bskip):
  del x
  ei = edge[0]
  src = ei[0].astype(jnp.int32)
  dst = ei[1].astype(jnp.int32)

  h, q, k, v, skip, a_src, a_dst, selfw = _dense_pre(
      embedding, W_gat, att_src, att_dst, Wq, bq, Wk, bk, Wv, bv, Wskip,
      bskip)

  w_gat, denp_g = _gat_edge_sc(a_src.reshape(N), a_dst.reshape(N), src, dst)
  tw, denp_t = _trans_edge_sc(q, k, src, dst)

  inv_g_row, inv_t_row = _combine(denp_g.reshape(NW, N), selfw.reshape(1, N), denp_t.reshape(NW, N))

  zeros_nd = jnp.zeros((N, D), jnp.float32)
  gat_p = _agg_sc(h, w_gat, src, dst, zeros_nd)
  t_p = _agg_sc(v, tw, src, dst, zeros_nd)

  return _final(gat_p, t_p, h, selfw, inv_g_row.reshape(N, 1),
                inv_t_row.reshape(N, 1), skip, b_gat)


# speedup vs baseline: 1.1989x; 1.1989x over previous
import functools
import math

import jax
import jax.numpy as jnp
from jax import lax
from jax.experimental import pallas as pl
from jax.experimental.pallas import tpu as pltpu
from jax.experimental.pallas import tpu_sc as plsc

N = 10000
D = 128
NC = 2
NS = 16
NW = NC * NS
L = 16
CH = 128

_mesh = functools.partial(
    plsc.VectorSubcoreMesh,
    core_axis_name="c", subcore_axis_name="s",
    num_cores=NC, num_subcores=NS,
)


def _wid():
  return lax.axis_index("s") * NC + lax.axis_index("c")


def _pre_body(emb, Wg, asr, adr, Wq, bq, Wk, bk, Wv, bv, Ws, bs,
              h_o, q_o, k_o, v_o, skip_o, a_src_o, a_dst_o, selfw_o):
  e = emb[...]
  h = jnp.dot(e, Wg[...], preferred_element_type=jnp.float32)
  h_o[...] = h
  a_s = jnp.sum(h * asr[...], axis=1, keepdims=True)
  a_d = jnp.sum(h * adr[...], axis=1, keepdims=True)
  a_src_o[...] = a_s
  a_dst_o[...] = a_d
  ls = a_s + a_d
  selfw_o[...] = jnp.exp(jnp.where(ls >= 0, ls, 0.2 * ls))
  q_o[...] = jnp.dot(e, Wq[...], preferred_element_type=jnp.float32) + bq[...]
  k_o[...] = jnp.dot(e, Wk[...], preferred_element_type=jnp.float32) + bk[...]
  v_o[...] = jnp.dot(e, Wv[...], preferred_element_type=jnp.float32) + bv[...]
  skip_o[...] = jnp.dot(e, Ws[...], preferred_element_type=jnp.float32) + bs[...]


def _dense_pre(emb, Wg, att_src, att_dst, Wq, bq, Wk, bk, Wv, bv, Ws, bs):
  R = 1000
  grid = (N // R,)
  row_blk = pl.BlockSpec((R, D), lambda i: (i, 0))
  full = pl.BlockSpec((D, D), lambda i: (0, 0))
  vec = pl.BlockSpec((1, D), lambda i: (0, 0))
  col = pl.BlockSpec((R, 1), lambda i: (i, 0))
  f32 = jnp.float32
  return pl.pallas_call(
      _pre_body,
      grid=grid,
      in_specs=[row_blk, full, vec, vec, full, vec, full, vec, full, vec,
                full, vec],
      out_specs=[row_blk, row_blk, row_blk, row_blk, row_blk, col, col, col],
      out_shape=[
          jax.ShapeDtypeStruct((N, D), f32),
          jax.ShapeDtypeStruct((N, D), f32),
          jax.ShapeDtypeStruct((N, D), f32),
          jax.ShapeDtypeStruct((N, D), f32),
          jax.ShapeDtypeStruct((N, D), f32),
          jax.ShapeDtypeStruct((N, 1), f32),
          jax.ShapeDtypeStruct((N, 1), f32),
          jax.ShapeDtypeStruct((N, 1), f32),
      ],
  )(emb, Wg, att_src.reshape(1, D), att_dst.reshape(1, D), Wq,
    bq.reshape(1, D), Wk, bk.reshape(1, D), Wv, bv.reshape(1, D), Ws,
    bs.reshape(1, D))


def _gat_edge_sc(a_src, a_dst, src, dst):
  E = src.shape[0]
  ept = E // NW
  f32 = jnp.float32

  @functools.partial(
      pl.kernel,
      out_type=(jax.ShapeDtypeStruct((E,), f32),
                jax.ShapeDtypeStruct((NW, 1, N), f32)),
      mesh=_mesh(),
      compiler_params=pltpu.CompilerParams(needs_layout_passes=False),
      scratch_types=[
          pltpu.VMEM((ept,), jnp.int32),
          pltpu.VMEM((ept,), jnp.int32),
          pltpu.VMEM((ept,), f32),
          pltpu.VMEM((N,), f32),
          pltpu.VMEM((N,), f32),
          pltpu.VMEM((N,), f32),
      ],
  )
  def k(a_src_h, a_dst_h, src_h, dst_h, w_h, denp_h,
        src_v, dst_v, w_v, as_v, ad_v, den_v):
    wid = _wid()
    base = wid * ept
    pltpu.sync_copy(src_h.at[pl.ds(base, ept)], src_v)
    pltpu.sync_copy(dst_h.at[pl.ds(base, ept)], dst_v)
    pltpu.sync_copy(a_src_h, as_v)
    pltpu.sync_copy(a_dst_h, ad_v)

    zero16 = jnp.zeros((L,), f32)

    def zero_body(i, _):
      den_v[pl.ds(i * L, L)] = zero16
      return 0
    lax.fori_loop(0, N // L, zero_body, 0)

    def body(g, _):
      s16 = src_v[pl.ds(g * L, L)]
      d16 = dst_v[pl.ds(g * L, L)]
      av = plsc.load_gather(as_v, [s16])
      bv = plsc.load_gather(ad_v, [d16])
      lg = av + bv
      lg = jnp.where(lg >= 0, lg, 0.2 * lg)
      wv = jnp.exp(lg)
      w_v[pl.ds(g * L, L)] = wv
      plsc.addupdate_scatter(den_v, [d16], wv)
      return 0
    lax.fori_loop(0, ept // L, body, 0)

    pltpu.sync_copy(w_v, w_h.at[pl.ds(base, ept)])
    pltpu.sync_copy(den_v, denp_h.at[wid, 0])

  return k(a_src, a_dst, src, dst)


def _trans_edge_sc(qp, kp, src, dst):
  E = src.shape[0]
  nch = E // CH
  base_cnt = nch // NW
  extra = nch - base_cnt * NW
  inv_sqrt_d = 1.0 / math.sqrt(D)
  f32 = jnp.float32

  NB = 3
  niter = (base_cnt + 1 + NB - 1) // NB

  @functools.partial(
      pl.kernel,
      out_type=(jax.ShapeDtypeStruct((E,), f32),
                jax.ShapeDtypeStruct((NW, 1, N), f32)),
      mesh=_mesh(),
      compiler_params=pltpu.CompilerParams(needs_layout_passes=False),
      scratch_types=[
          pltpu.VMEM((NB, CH), jnp.int32),
          pltpu.VMEM((NB, CH), jnp.int32),
          pltpu.VMEM((NB, CH, D), f32),
          pltpu.VMEM((NB, CH, D), f32),
          pltpu.VMEM((CH,), f32),
          pltpu.VMEM((N,), f32),
          pltpu.SemaphoreType.DMA,
          pltpu.SemaphoreType.DMA,
          pltpu.SemaphoreType.DMA,
      ],
  )
  def k(qp_h, kp_h, src_h, dst_h, tw_h, denp_h,
        si_v, di_v, qr_v, kr_v, tw_v, den_v, sem0, sem1, sem2):
    sems = (sem0, sem1, sem2)
    wid = _wid()
    zero16 = jnp.zeros((L,), f32)

    def zero_body(i, _):
      den_v[pl.ds(i * L, L)] = zero16
      return 0
    lax.fori_loop(0, N // L, zero_body, 0)

    my_cnt = base_cnt + jnp.where(wid < extra, 1, 0)
    iota = lax.iota(jnp.int32, L)

    def issue(c, b):
      @pl.when(c < my_cnt)
      def _():
        cb = (wid + c * NW) * CH
        pltpu.sync_copy(src_h.at[pl.ds(cb, CH)], si_v.at[b])
        pltpu.sync_copy(dst_h.at[pl.ds(cb, CH)], di_v.at[b])
        pltpu.async_copy(qp_h.at[di_v.at[b]], qr_v.at[b], sems[b])
        pltpu.async_copy(kp_h.at[si_v.at[b]], kr_v.at[b], sems[b])

    def process(c, b):
      @pl.when(c < my_cnt)
      def _():
        cb = (wid + c * NW) * CH
        pltpu.make_async_copy(qp_h.at[di_v.at[b]], qr_v.at[b], sems[b]).wait()
        pltpu.make_async_copy(kp_h.at[si_v.at[b]], kr_v.at[b], sems[b]).wait()

        def group_body(g, _):
          tlv = jnp.zeros((L,), f32)
          for e in range(L):
            row = g * L + e
            acc = jnp.zeros((L,), f32)
            for j in range(D // L):
              sl = pl.ds(j * L, L)
              acc = acc + qr_v[b, row, sl] * kr_v[b, row, sl]
            tlv = jnp.where(iota == e, jnp.sum(acc), tlv)
          wv = jnp.exp(tlv * inv_sqrt_d)
          tw_v[pl.ds(g * L, L)] = wv
          d16 = di_v[b, pl.ds(g * L, L)]
          plsc.addupdate_scatter(den_v, [d16], wv)
          return 0
        lax.fori_loop(0, CH // L, group_body, 0)
        pltpu.sync_copy(tw_v, tw_h.at[pl.ds(cb, CH)])

    for b in range(NB - 1):
      issue(b, b)

    def chunk_body(t, _):
      for b in range(NB):
        c = t * NB + b
        issue(c + NB - 1, (b + NB - 1) % NB)
        process(c, b)
      return 0
    lax.fori_loop(0, niter, chunk_body, 0)

    pltpu.sync_copy(den_v, denp_h.at[wid, 0])

  return k(qp, kp, src, dst)


def _agg_sc(rows, w, src, dst, zeros_nd):
  E = src.shape[0]
  CHA = 128
  nch = E // CHA
  base_cnt = nch // NW
  extra = nch - base_cnt * NW
  NB = 3
  niter = (base_cnt + 1 + NB - 1) // NB
  RB = 80
  nrb = N // RB
  rb_base = nrb // NS
  rb_extra = nrb - rb_base * NS
  f32 = jnp.float32

  @functools.partial(
      pl.kernel,
      out_type=jax.ShapeDtypeStruct((NC, N, D), f32),
      mesh=_mesh(),
      compiler_params=pltpu.CompilerParams(needs_layout_passes=False),
      scratch_types=[
          pltpu.VMEM((NB, CHA), jnp.int32),
          pltpu.VMEM((NB, CHA), jnp.int32),
          pltpu.VMEM((NB, CHA, D), f32),
          pltpu.VMEM((NB, CHA), f32),
          pltpu.VMEM_SHARED((N, D), f32),
          pltpu.SemaphoreType.DMA,
          pltpu.SemaphoreType.DMA,
          pltpu.SemaphoreType.DMA,
      ],
  )
  def k(rows_h, w_h, src_h, dst_h, zeros_h, out_h,
        si_v, di_v, r_v, w_v, acc, sem0, sem1, sem2):
    sems = (sem0, sem1, sem2)
    cid = lax.axis_index("c")
    sid = lax.axis_index("s")
    wid = sid * NC + cid

    def init_body(ci, _):
      rb = (sid + ci * NS) * RB
      pltpu.sync_copy(zeros_h.at[pl.ds(rb, RB)], acc.at[pl.ds(rb, RB)])
      return 0
    lax.fori_loop(0, rb_base + jnp.where(sid < rb_extra, 1, 0), init_body, 0)
    plsc.subcore_barrier()

    my_cnt = base_cnt + jnp.where(wid < extra, 1, 0)

    def issue(c, b):
      @pl.when(c < my_cnt)
      def _():
        cb = (wid + c * NW) * CHA
        pltpu.sync_copy(src_h.at[pl.ds(cb, CHA)], si_v.at[b])
        pltpu.sync_copy(dst_h.at[pl.ds(cb, CHA)], di_v.at[b])
        pltpu.sync_copy(w_h.at[pl.ds(cb, CHA)], w_v.at[b])
        pltpu.async_copy(rows_h.at[si_v.at[b]], r_v.at[b], sems[b])

    def process(c, b):
      @pl.when(c < my_cnt)
      def _():
        pltpu.make_async_copy(rows_h.at[si_v.at[b]], r_v.at[b],
                              sems[b]).wait()

        def scale_body(g, _):
          w16 = w_v[b, pl.ds(g * L, L)]
          for e in range(L):
            a = w16[e]
            row = g * L + e
            for j in range(D // L):
              sl = pl.ds(j * L, L)
              r_v[b, row, sl] = r_v[b, row, sl] * a
          return 0
        lax.fori_loop(0, CHA // L, scale_body, 0)

        pltpu.sync_copy(r_v.at[b], acc.at[di_v.at[b]], add=True)

    for b in range(NB - 1):
      issue(b, b)

    def chunk_body(t, _):
      for b in range(NB):
        c = t * NB + b
        issue(c + NB - 1, (b + NB - 1) % NB)
        process(c, b)
      return 0
    lax.fori_loop(0, niter, chunk_body, 0)

    plsc.subcore_barrier()

    def flush_body(ci, _):
      rb = (sid + ci * NS) * RB
      pltpu.sync_copy(acc.at[pl.ds(rb, RB)], out_h.at[cid].at[pl.ds(rb, RB)])
      return 0
    lax.fori_loop(0, rb_base + jnp.where(sid < rb_extra, 1, 0), flush_body, 0)

  return k(rows, w, src, dst, zeros_nd)


def _combine_body(dg, selfw, dt, inv_g_o, inv_t_o):
  sg = jnp.sum(dg[...], axis=0, keepdims=True) + selfw[...]
  st = jnp.sum(dt[...], axis=0, keepdims=True)
  inv_g_o[...] = 1.0 / (sg + 1e-16)
  inv_t_o[...] = 1.0 / (st + 1e-16)


def _combine(denp_g, selfw_row, denp_t):
  f32 = jnp.float32
  return pl.pallas_call(
      _combine_body,
      out_shape=[jax.ShapeDtypeStruct((1, N), f32),
                 jax.ShapeDtypeStruct((1, N), f32)],
  )(denp_g, selfw_row, denp_t)


def _final_body(gp, tp, h, selfw, inv_g, inv_t, skip, bg, out_o):
  inv_g = inv_g[...]
  inv_t = inv_t[...]
  gat = (gp[0] + gp[1] + h[...] * selfw[...]) * inv_g + bg[...]
  x1 = jnp.maximum(gat, 0.0)
  tout = (tp[0] + tp[1]) * inv_t + skip[...]
  out = x1 + tout
  nrm = jnp.sqrt(jnp.sum(out * out, axis=1, keepdims=True))
  out_o[...] = out / jnp.maximum(nrm, 1e-12)


def _final(gat_p, t_p, h, selfw, inv_g, inv_t, skip, b_gat):
  R = 1000
  f32 = jnp.float32
  part_blk = pl.BlockSpec((NC, R, D), lambda i: (0, i, 0))
  row_blk = pl.BlockSpec((R, D), lambda i: (i, 0))
  col_blk = pl.BlockSpec((R, 1), lambda i: (i, 0))
  vec = pl.BlockSpec((1, D), lambda i: (0, 0))
  return pl.pallas_call(
      _final_body,
      grid=(N // R,),
      in_specs=[part_blk, part_blk, row_blk, col_blk, col_blk, col_blk,
                row_blk, vec],
      out_specs=row_blk,
      out_shape=jax.ShapeDtypeStruct((N, D), f32),
  )(gat_p, t_p, h, selfw, inv_g, inv_t, skip, b_gat.reshape(1, D))


@jax.jit
def kernel(x, edge, embedding, W_gat, att_src, att_dst, b_gat,
           Wq, bq, Wk, bk, Wv, bv, Wskip, bskip):
  del x
  ei = edge[0]
  src = ei[0].astype(jnp.int32)
  dst = ei[1].astype(jnp.int32)

  h, q, k, v, skip, a_src, a_dst, selfw = _dense_pre(
      embedding, W_gat, att_src, att_dst, Wq, bq, Wk, bk, Wv, bv, Wskip,
      bskip)

  w_gat, denp_g = _gat_edge_sc(a_src.reshape(N), a_dst.reshape(N), src, dst)
  tw, denp_t = _trans_edge_sc(q, k, src, dst)

  inv_g_row, inv_t_row = _combine(denp_g.reshape(NW, N),
                                  selfw.reshape(1, N),
                                  denp_t.reshape(NW, N))

  zeros_nd = jnp.zeros((N, D), jnp.float32)
  gat_p = _agg_sc(h, w_gat, src, dst, zeros_nd)
  t_p = _agg_sc(v, tw, src, dst, zeros_nd)

  return _final(gat_p, t_p, h, selfw, inv_g_row.reshape(N, 1),
                inv_t_row.reshape(N, 1), skip, b_gat)
